# baseline (speedup 1.0000x reference)
import functools

import jax
import jax.numpy as jnp
from jax import lax
from jax.experimental import pallas as pl
from jax.experimental.pallas import tpu as pltpu
from jax.experimental.pallas import tpu_sc as plsc

N = 10000
E = 320000
D = 128
C = 64

NC = 2
NS = 16
EPW = E // (NC * NS)
CHUNK = 100
NCHUNK = EPW // CHUNK
IB = 5
NBLK = NCHUNK // IB
N_PAD = 10240
ROWS_PER_SUB = N_PAD // NS
ZR = 64


def _sc_scatter_partials(h, idx6):
  mesh = plsc.VectorSubcoreMesh(core_axis_name="c", subcore_axis_name="s",
                                num_cores=NC, num_subcores=NS)

  @functools.partial(
      pl.kernel,
      out_type=jax.ShapeDtypeStruct((NC, N_PAD, D), jnp.float32),
      mesh=mesh,
      scratch_types=[
          pltpu.VMEM((2, 2, IB, CHUNK), jnp.int32),
          pltpu.VMEM((3, CHUNK, D), jnp.float32),
          pltpu.VMEM_SHARED((N_PAD, D), jnp.float32),
          pltpu.SemaphoreType.DMA,
          pltpu.SemaphoreType.DMA,
          pltpu.SemaphoreType.DMA,
          pltpu.SemaphoreType.DMA,
          pltpu.SemaphoreType.DMA,
          pltpu.SemaphoreType.DMA,
          pltpu.SemaphoreType.DMA,
          pltpu.SemaphoreType.DMA,
      ],
  )
  def k(h_hbm, idx_hbm, out_hbm,
        idx_v, rows_v, acc_sh, gs0, gs1, gs2, ss0, ss1, ss2, is0, is1):
    c = lax.axis_index("c")
    s = lax.axis_index("s")
    row0 = s * ROWS_PER_SUB

    @pl.loop(0, ZR)
    def _(r):
      @pl.loop(0, D, step=16)
      def _(c0):
        rows_v[0, r, pl.ds(c0, 16)] = jnp.zeros((16,), jnp.float32)

    @pl.loop(0, ROWS_PER_SUB, step=2 * ZR)
    def _(r):
      pltpu.async_copy(rows_v.at[0, pl.ds(0, ZR)],
                       acc_sh.at[pl.ds(row0 + r, ZR)], ss2)
      pltpu.async_copy(rows_v.at[0, pl.ds(0, ZR)],
                       acc_sh.at[pl.ds(row0 + r + ZR, ZR)], ss2)

    @pl.loop(0, ROWS_PER_SUB, step=ZR)
    def _(r):
      pltpu.make_async_copy(h_hbm.at[pl.ds(0, ZR)],
                            acc_sh.at[pl.ds(row0, ZR)], ss2).wait()

    plsc.subcore_barrier()

    gsems = (gs0, gs1, gs2)
    ssems = (ss0, ss1, ss2)

    @pl.loop(0, NBLK, step=2)
    def _(blk):
      dA = pltpu.async_copy(idx_hbm.at[c, s, blk], idx_v.at[0], is0)
      dB = pltpu.async_copy(idx_hbm.at[c, s, blk + 1], idx_v.at[1], is1)
      dA.wait()

      def gref(jj):
        p, j = divmod(jj, IB)
        return idx_v.at[p, 0, j]

      def sref(jj):
        p, j = divmod(jj, IB)
        return idx_v.at[p, 1, j]

      TB = 2 * IB
      gd = {}
      sd = {}
      waited = set()
      for j in range(2):
        gd[j] = pltpu.async_copy(h_hbm.at[gref(j)], rows_v.at[j], gsems[j])
      for j in range(TB):
        b = j % 3
        gd[j].wait()
        sd[j] = pltpu.async_copy(rows_v.at[b], acc_sh.at[sref(j)],
                                 ssems[b], add=True)
        nj = j + 1
        if nj < TB:
          if nj - 3 >= 0:
            sd[nj - 3].wait()
            waited.add(nj - 3)
          if nj == IB:
            dB.wait()
          gd[nj] = pltpu.async_copy(h_hbm.at[gref(nj)],
                                    rows_v.at[nj % 3], gsems[nj % 3])
      for j in range(TB):
        if j not in waited:
          sd[j].wait()

    plsc.subcore_barrier()
    pltpu.sync_copy(acc_sh.at[pl.ds(row0, ROWS_PER_SUB)],
                    out_hbm.at[c].at[pl.ds(row0, ROWS_PER_SUB)])

  return k(h, idx6)


_R = 2000


def _l2n(h):
  n = jnp.sqrt(jnp.sum(h * h, axis=1, keepdims=True))
  return h / jnp.maximum(n, 1e-12)


def _pre_body(x_ref, w_ref, b_ref, o_ref):
  h = jnp.dot(x_ref[...], w_ref[...], preferred_element_type=jnp.float32)
  h = jnp.maximum(h + b_ref[...], 0.0)
  o_ref[...] = _l2n(h)


def _tc_pre(x, W, b):
  return pl.pallas_call(
      _pre_body,
      grid=(N // _R,),
      in_specs=[
          pl.BlockSpec((_R, D), lambda i: (i, 0)),
          pl.BlockSpec((D, D), lambda i: (0, 0)),
          pl.BlockSpec((1, D), lambda i: (0, 0)),
      ],
      out_specs=pl.BlockSpec((_R, D), lambda i: (i, 0)),
      out_shape=jax.ShapeDtypeStruct((N, D), jnp.float32),
  )(x, W, b.reshape(1, D))


def _root_body(h_ref, wr_ref, b_ref, o_ref):
  o_ref[...] = (jnp.dot(h_ref[...], wr_ref[...],
                        preferred_element_type=jnp.float32) + b_ref[...])


def _tc_root(h, Wr, b):
  return pl.pallas_call(
      _root_body,
      grid=(N // _R,),
      in_specs=[
          pl.BlockSpec((_R, D), lambda i: (i, 0)),
          pl.BlockSpec((D, D), lambda i: (0, 0)),
          pl.BlockSpec((1, D), lambda i: (0, 0)),
      ],
      out_specs=pl.BlockSpec((_R, D), lambda i: (i, 0)),
      out_shape=jax.ShapeDtypeStruct((N, D), jnp.float32),
  )(h, Wr, b.reshape(1, D))


def _combine_body(p_ref, hr_ref, wl_ref, o_ref):
  agg = p_ref[0] + p_ref[1]
  z = (jnp.dot(agg, wl_ref[...], preferred_element_type=jnp.float32)
       + hr_ref[...])
  o_ref[...] = jnp.maximum(_l2n(z), 0.0)


def _tc_combine(p, hr, Wl):
  return pl.pallas_call(
      _combine_body,
      grid=(N // _R,),
      in_specs=[
          pl.BlockSpec((NC, _R, D), lambda i: (0, i, 0)),
          pl.BlockSpec((_R, D), lambda i: (i, 0)),
          pl.BlockSpec((D, D), lambda i: (0, 0)),
      ],
      out_specs=pl.BlockSpec((_R, D), lambda i: (i, 0)),
      out_shape=jax.ShapeDtypeStruct((N, D), jnp.float32),
  )(p, hr, Wl)


def _final_body(p_ref, hr_ref, wl_ref, wp_ref, bp_ref, o_ref):
  agg = p_ref[0] + p_ref[1]
  z = (jnp.dot(agg, wl_ref[...], preferred_element_type=jnp.float32)
       + hr_ref[...])
  h2 = jnp.maximum(_l2n(z), 0.0)
  logits = (jnp.dot(h2, wp_ref[...], preferred_element_type=jnp.float32)
            + bp_ref[...])
  m = jnp.max(logits, axis=1, keepdims=True)
  lse = m + jnp.log(jnp.sum(jnp.exp(logits - m), axis=1, keepdims=True))
  o_ref[...] = logits - lse


def _tc_final(p, hr, Wl, Wp, bp):
  return pl.pallas_call(
      _final_body,
      grid=(N // _R,),
      in_specs=[
          pl.BlockSpec((NC, _R, D), lambda i: (0, i, 0)),
          pl.BlockSpec((_R, D), lambda i: (i, 0)),
          pl.BlockSpec((D, D), lambda i: (0, 0)),
          pl.BlockSpec((D, C), lambda i: (0, 0)),
          pl.BlockSpec((1, C), lambda i: (0, 0)),
      ],
      out_specs=pl.BlockSpec((_R, C), lambda i: (i, 0)),
      out_shape=jax.ShapeDtypeStruct((N, C), jnp.float32),
  )(p, hr, Wl, Wp, bp.reshape(1, C))


def kernel(x, edge_index, W_pre, b_pre, Wl1, Wr1, b1, Wl2, Wr2, b2,
           W_post, b_post):
  idx6 = jnp.transpose(
      edge_index.astype(jnp.int32).reshape(2, NC, NS, NBLK, IB, CHUNK),
      (1, 2, 3, 0, 4, 5))

  h = _tc_pre(x, W_pre, b_pre)
  p1 = _sc_scatter_partials(h, idx6)
  hr1 = _tc_root(h, Wr1, b1)
  h1 = _tc_combine(p1, hr1, Wl1)
  p2 = _sc_scatter_partials(h1, idx6)
  hr2 = _tc_root(h1, Wr2, b2)
  return _tc_final(p2, hr2, Wl2, W_post, b_post)


# speedup vs baseline: 1.0098x; 1.0098x over previous
import functools

import jax
import jax.numpy as jnp
from jax import lax
from jax.experimental import pallas as pl
from jax.experimental.pallas import tpu as pltpu
from jax.experimental.pallas import tpu_sc as plsc

N = 10000
E = 320000
D = 128
C = 64

NC = 2
NS = 16
EPW = E // (NC * NS)
CHUNK = 100
NCHUNK = EPW // CHUNK
IB = 5
NBLK = NCHUNK // IB
N_PAD = 10240
ROWS_PER_SUB = N_PAD // NS
ZR = 64


def _sc_scatter_partials(h, idx6):
  mesh = plsc.VectorSubcoreMesh(core_axis_name="c", subcore_axis_name="s",
                                num_cores=NC, num_subcores=NS)

  @functools.partial(
      pl.kernel,
      out_type=jax.ShapeDtypeStruct((NC, N_PAD, D), jnp.float32),
      mesh=mesh,
      scratch_types=[
          pltpu.VMEM((2, 2, IB, CHUNK), jnp.int32),
          pltpu.VMEM((3, CHUNK, D), jnp.float32),
          pltpu.VMEM_SHARED((N_PAD, D), jnp.float32),
          pltpu.SemaphoreType.DMA,
          pltpu.SemaphoreType.DMA,
          pltpu.SemaphoreType.DMA,
          pltpu.SemaphoreType.DMA,
          pltpu.SemaphoreType.DMA,
          pltpu.SemaphoreType.DMA,
          pltpu.SemaphoreType.DMA,
          pltpu.SemaphoreType.DMA,
      ],
  )
  def k(h_hbm, idx_hbm, out_hbm,
        idx_v, rows_v, acc_sh, gs0, gs1, gs2, ss0, ss1, ss2, is0, is1):
    c = lax.axis_index("c")
    s = lax.axis_index("s")
    row0 = s * ROWS_PER_SUB

    @pl.loop(0, ZR)
    def _(r):
      @pl.loop(0, D, step=16)
      def _(c0):
        rows_v[0, r, pl.ds(c0, 16)] = jnp.zeros((16,), jnp.float32)

    @pl.loop(0, ROWS_PER_SUB, step=2 * ZR)
    def _(r):
      pltpu.async_copy(rows_v.at[0, pl.ds(0, ZR)],
                       acc_sh.at[pl.ds(row0 + r, ZR)], ss2)
      pltpu.async_copy(rows_v.at[0, pl.ds(0, ZR)],
                       acc_sh.at[pl.ds(row0 + r + ZR, ZR)], ss2)

    @pl.loop(0, ROWS_PER_SUB, step=ZR)
    def _(r):
      pltpu.make_async_copy(h_hbm.at[pl.ds(0, ZR)],
                            acc_sh.at[pl.ds(row0, ZR)], ss2).wait()

    plsc.subcore_barrier()

    gsems = (gs0, gs1, gs2)
    ssems = (ss0, ss1, ss2)

    @pl.loop(0, NBLK, step=2)
    def _(blk):
      dA = pltpu.async_copy(idx_hbm.at[c, s, blk], idx_v.at[0], is0)
      dB = pltpu.async_copy(idx_hbm.at[c, s, blk + 1], idx_v.at[1], is1)
      dA.wait()

      def gref(jj):
        p, j = divmod(jj, IB)
        return idx_v.at[p, 0, j]

      def sref(jj):
        p, j = divmod(jj, IB)
        return idx_v.at[p, 1, j]

      TB = 2 * IB
      gd = {}
      sd = {}
      waited = set()
      for j in range(2):
        gd[j] = pltpu.async_copy(h_hbm.at[gref(j)], rows_v.at[j], gsems[j])
      for j in range(TB):
        b = j % 3
        gd[j].wait()
        sd[j] = pltpu.async_copy(rows_v.at[b], acc_sh.at[sref(j)],
                                 ssems[b], add=True)
        nj = j + 1
        if nj < TB:
          if nj - 3 >= 0:
            sd[nj - 3].wait()
            waited.add(nj - 3)
          if nj == IB:
            dB.wait()
          gd[nj] = pltpu.async_copy(h_hbm.at[gref(nj)],
                                    rows_v.at[nj % 3], gsems[nj % 3])
      for j in range(TB):
        if j not in waited:
          sd[j].wait()

    plsc.subcore_barrier()
    pltpu.sync_copy(acc_sh.at[pl.ds(row0, ROWS_PER_SUB)],
                    out_hbm.at[c].at[pl.ds(row0, ROWS_PER_SUB)])

  return k(h, idx6)


_R = 2000


def _l2n(h):
  n = jnp.sqrt(jnp.sum(h * h, axis=1, keepdims=True))
  return h / jnp.maximum(n, 1e-12)


def _pre_body(x_ref, w_ref, b_ref, o_ref):
  h = jnp.dot(x_ref[...], w_ref[...], preferred_element_type=jnp.float32)
  h = jnp.maximum(h + b_ref[...], 0.0)
  o_ref[...] = _l2n(h)


def _tc_pre(x, W, b):
  return pl.pallas_call(
      _pre_body,
      grid=(N // _R,),
      in_specs=[
          pl.BlockSpec((_R, D), lambda i: (i, 0)),
          pl.BlockSpec((D, D), lambda i: (0, 0)),
          pl.BlockSpec((1, D), lambda i: (0, 0)),
      ],
      out_specs=pl.BlockSpec((_R, D), lambda i: (i, 0)),
      out_shape=jax.ShapeDtypeStruct((N, D), jnp.float32),
  )(x, W, b.reshape(1, D))


def _combine_body(p_ref, h_ref, wl_ref, wr_ref, b_ref, o_ref):
  agg = p_ref[0] + p_ref[1]
  z = (jnp.dot(agg, wl_ref[...], preferred_element_type=jnp.float32)
       + jnp.dot(h_ref[...], wr_ref[...], preferred_element_type=jnp.float32)
       + b_ref[...])
  o_ref[...] = jnp.maximum(_l2n(z), 0.0)


def _tc_combine(p, h, Wl, Wr, b):
  return pl.pallas_call(
      _combine_body,
      grid=(N // _R,),
      in_specs=[
          pl.BlockSpec((NC, _R, D), lambda i: (0, i, 0)),
          pl.BlockSpec((_R, D), lambda i: (i, 0)),
          pl.BlockSpec((D, D), lambda i: (0, 0)),
          pl.BlockSpec((D, D), lambda i: (0, 0)),
          pl.BlockSpec((1, D), lambda i: (0, 0)),
      ],
      out_specs=pl.BlockSpec((_R, D), lambda i: (i, 0)),
      out_shape=jax.ShapeDtypeStruct((N, D), jnp.float32),
  )(p, h, Wl, Wr, b.reshape(1, D))


def _final_body(p_ref, h_ref, wl_ref, wr_ref, b_ref, wp_ref, bp_ref, o_ref):
  agg = p_ref[0] + p_ref[1]
  z = (jnp.dot(agg, wl_ref[...], preferred_element_type=jnp.float32)
       + jnp.dot(h_ref[...], wr_ref[...], preferred_element_type=jnp.float32)
       + b_ref[...])
  h2 = jnp.maximum(_l2n(z), 0.0)
  logits = (jnp.dot(h2, wp_ref[...], preferred_element_type=jnp.float32)
            + bp_ref[...])
  m = jnp.max(logits, axis=1, keepdims=True)
  lse = m + jnp.log(jnp.sum(jnp.exp(logits - m), axis=1, keepdims=True))
  o_ref[...] = logits - lse


def _tc_final(p, h1, Wl, Wr, b, Wp, bp):
  return pl.pallas_call(
      _final_body,
      grid=(N // _R,),
      in_specs=[
          pl.BlockSpec((NC, _R, D), lambda i: (0, i, 0)),
          pl.BlockSpec((_R, D), lambda i: (i, 0)),
          pl.BlockSpec((D, D), lambda i: (0, 0)),
          pl.BlockSpec((D, D), lambda i: (0, 0)),
          pl.BlockSpec((1, D), lambda i: (0, 0)),
          pl.BlockSpec((D, C), lambda i: (0, 0)),
          pl.BlockSpec((1, C), lambda i: (0, 0)),
      ],
      out_specs=pl.BlockSpec((_R, C), lambda i: (i, 0)),
      out_shape=jax.ShapeDtypeStruct((N, C), jnp.float32),
  )(p, h1, Wl, Wr, b.reshape(1, D), Wp, bp.reshape(1, C))


def kernel(x, edge_index, W_pre, b_pre, Wl1, Wr1, b1, Wl2, Wr2, b2,
           W_post, b_post):
  idx6 = jnp.transpose(
      edge_index.astype(jnp.int32).reshape(2, NC, NS, NBLK, IB, CHUNK),
      (1, 2, 3, 0, 4, 5))

  h = _tc_pre(x, W_pre, b_pre)
  p1 = _sc_scatter_partials(h, idx6)
  h1 = _tc_combine(p1, h, Wl1, Wr1, b1)
  p2 = _sc_scatter_partials(h1, idx6)
  return _tc_final(p2, h1, Wl2, Wr2, b2, W_post, b_post)
